# Initial kernel scaffold; baseline (speedup 1.0000x reference)
import jax
import jax.numpy as jnp
from jax.experimental import pallas as pl


def kernel(x, edge_index, edge_attr, eb_W0, eb_b0, eb_W1, eb_b1, eb_W2, eb_b2, eb_W3, eb_b3, eb_g, eb_beta, nb_W0, nb_b0, nb_W1, nb_b1, nb_W2, nb_b2, nb_W3, nb_b3, nb_g, nb_beta):
    raise NotImplementedError("write your pallas kernel here")


# speedup vs baseline: 1.6069x; 1.6069x over previous
import functools

import jax
import jax.numpy as jnp
from jax import lax
from jax.experimental import pallas as pl
from jax.experimental.pallas import tpu as pltpu
from jax.experimental.pallas import tpu_sc as plsc

_PREC = lax.Precision.HIGHEST


def _tables_body(x_ref, ws_ref, wr_ref, ts_ref, tr_ref):
    xb = x_ref[...]
    ts_ref[...] = jnp.dot(xb, ws_ref[...], preferred_element_type=jnp.float32,
                          precision=_PREC)
    tr_ref[...] = jnp.dot(xb, wr_ref[...], preferred_element_type=jnp.float32,
                          precision=_PREC)


def _make_tables(x, ws, wr):
    n, h = x.shape
    tb = 2000
    return pl.pallas_call(
        _tables_body,
        grid=(n // tb,),
        in_specs=[
            pl.BlockSpec((tb, h), lambda i: (i, 0)),
            pl.BlockSpec((h, h), lambda i: (0, 0)),
            pl.BlockSpec((h, h), lambda i: (0, 0)),
        ],
        out_specs=[
            pl.BlockSpec((tb, h), lambda i: (i, 0)),
            pl.BlockSpec((tb, h), lambda i: (i, 0)),
        ],
        out_shape=[jax.ShapeDtypeStruct((n, h), jnp.float32)] * 2,
    )(x, ws, wr)


_GW = 80


def _sc_gather(table, idx):
    n, h = table.shape
    e = idx.shape[0]
    mesh = plsc.VectorSubcoreMesh(core_axis_name="core",
                                  subcore_axis_name="subcore")

    @functools.partial(
        pl.kernel,
        out_type=jax.ShapeDtypeStruct((e, h), jnp.float32),
        mesh=mesh,
    )
    def k(t_hbm, i_hbm, o_hbm):
        def body(i_vmem, o_vmem):
            pltpu.sync_copy(t_hbm.at[i_vmem.at[0, 0]], o_vmem)

        pltpu.emit_pipeline(
            body,
            grid=(e // _GW,),
            in_specs=[pl.BlockSpec((1, 1, _GW), lambda i: (i, 0, 0))],
            out_specs=[pl.BlockSpec((_GW, h), lambda i: (i, 0))],
            core_axis_name=("core", "subcore"),
            dimension_semantics=(pltpu.PARALLEL,),
        )(i_hbm, o_hbm)

    return k(table, idx.reshape(e // _GW, 1, _GW))


def _sc_scatter(en, receivers, n):
    e, h = en.shape
    n_sub = 16
    n_pad = 10240
    rows_per_sub = n_pad // n_sub
    zb = 128
    mesh = plsc.VectorSubcoreMesh(core_axis_name="core",
                                  subcore_axis_name="subcore")

    @functools.partial(
        pl.kernel,
        out_type=jax.ShapeDtypeStruct((2 * n_pad, h), jnp.float32),
        mesh=mesh,
        scratch_types=[
            pltpu.VMEM((zb, h), jnp.float32),
            pltpu.VMEM_SHARED((n_pad, h), jnp.float32),
        ],
    )
    def k(en_hbm, r_hbm, out_hbm, zbuf, agg_sh):
        cid = lax.axis_index("core")
        sid = lax.axis_index("subcore")

        @pl.loop(0, zb)
        def _(rr):
            for j in range(h // 16):
                zbuf.at[pl.ds(rr, 1), pl.ds(j * 16, 16)][...] = (
                    jnp.zeros((1, 16), jnp.float32))

        @pl.loop(0, rows_per_sub // zb)
        def _(kk):
            pltpu.sync_copy(
                zbuf, agg_sh.at[pl.ds(sid * rows_per_sub + kk * zb, zb)])

        plsc.subcore_barrier()

        def body(en_vmem, r_vmem):
            pltpu.sync_copy(en_vmem, agg_sh.at[r_vmem.at[0, 0]], add=True)

        pltpu.emit_pipeline(
            body,
            grid=(e // _GW,),
            in_specs=[pl.BlockSpec((_GW, h), lambda i: (i, 0)),
                      pl.BlockSpec((1, 1, _GW), lambda i: (i, 0, 0))],
            out_specs=[],
            core_axis_name=("core", "subcore"),
            dimension_semantics=(pltpu.PARALLEL,),
        )(en_hbm, r_hbm)

        plsc.subcore_barrier()

        @pl.loop(0, rows_per_sub // zb)
        def _(kk):
            pltpu.sync_copy(
                agg_sh.at[pl.ds(sid * rows_per_sub + kk * zb, zb)], zbuf)
            pltpu.sync_copy(
                zbuf,
                out_hbm.at[
                    pl.ds(cid * n_pad + sid * rows_per_sub + kk * zb, zb)])

    return k(en, receivers.reshape(e // _GW, 1, _GW)), n_pad


def _edge_body(gs_ref, gr_ref, attr_ref, w0e, b0, w1, b1, w2, b2, w3, b3,
               g, beta, en_ref, eo_ref):
    attr = attr_ref[...]
    h = (gs_ref[...] + gr_ref[...] + b0[...]
         + jnp.dot(attr, w0e[...], preferred_element_type=jnp.float32,
                   precision=_PREC))
    h = jnp.maximum(h, 0.0)
    h = jnp.maximum(
        jnp.dot(h, w1[...], preferred_element_type=jnp.float32,
                precision=_PREC) + b1[...], 0.0)
    h = jnp.maximum(
        jnp.dot(h, w2[...], preferred_element_type=jnp.float32,
                precision=_PREC) + b2[...], 0.0)
    h = jnp.dot(h, w3[...], preferred_element_type=jnp.float32,
                precision=_PREC) + b3[...]
    mu = jnp.mean(h, axis=-1, keepdims=True)
    d = h - mu
    var = jnp.mean(d * d, axis=-1, keepdims=True)
    en = (d * lax.rsqrt(var + 1e-5)) * g[...] + beta[...]
    en_ref[...] = en
    eo_ref[...] = attr + en


def _edge_mlp(gs, gr, attr, w0e, b0, w1, b1, w2, b2, w3, b3, g, beta):
    e, h = attr.shape
    te = 512
    row = lambda i: (i, 0)
    whole = lambda i: (0, 0)
    wspec = pl.BlockSpec((h, h), whole)
    bspec = pl.BlockSpec((1, h), whole)
    return pl.pallas_call(
        _edge_body,
        grid=(e // te,),
        in_specs=[pl.BlockSpec((te, h), row)] * 3
        + [wspec, bspec, wspec, bspec, wspec, bspec, wspec, bspec,
           bspec, bspec],
        out_specs=[pl.BlockSpec((te, h), row)] * 2,
        out_shape=[jax.ShapeDtypeStruct((e, h), jnp.float32)] * 2,
    )(gs, gr, attr, w0e, b0, w1, b1, w2, b2, w3, b3, g, beta)


def _node_body(x_ref, p0_ref, p1_ref, wx, wa, b0, w1, b1, w2, b2, w3, b3,
               g, beta, xo_ref):
    xb = x_ref[...]
    agg = p0_ref[...] + p1_ref[...]
    h = (jnp.dot(xb, wx[...], preferred_element_type=jnp.float32,
                 precision=_PREC)
         + jnp.dot(agg, wa[...], preferred_element_type=jnp.float32,
                   precision=_PREC) + b0[...])
    h = jnp.maximum(h, 0.0)
    h = jnp.maximum(
        jnp.dot(h, w1[...], preferred_element_type=jnp.float32,
                precision=_PREC) + b1[...], 0.0)
    h = jnp.maximum(
        jnp.dot(h, w2[...], preferred_element_type=jnp.float32,
                precision=_PREC) + b2[...], 0.0)
    h = jnp.dot(h, w3[...], preferred_element_type=jnp.float32,
                precision=_PREC) + b3[...]
    mu = jnp.mean(h, axis=-1, keepdims=True)
    d = h - mu
    var = jnp.mean(d * d, axis=-1, keepdims=True)
    xo_ref[...] = xb + (d * lax.rsqrt(var + 1e-5)) * g[...] + beta[...]


def _node_mlp(x, p0, p1, wx, wa, b0, w1, b1, w2, b2, w3, b3, g, beta):
    n, h = x.shape
    tn = 1000
    row = lambda i: (i, 0)
    whole = lambda i: (0, 0)
    wspec = pl.BlockSpec((h, h), whole)
    bspec = pl.BlockSpec((1, h), whole)
    return pl.pallas_call(
        _node_body,
        grid=(n // tn,),
        in_specs=[pl.BlockSpec((tn, h), row)] * 3
        + [wspec, wspec, bspec, wspec, bspec, wspec, bspec, wspec, bspec,
           bspec, bspec],
        out_specs=pl.BlockSpec((tn, h), row),
        out_shape=jax.ShapeDtypeStruct((n, h), jnp.float32),
    )(x, p0, p1, wx, wa, b0, w1, b1, w2, b2, w3, b3, g, beta)


def kernel(x, edge_index, edge_attr, eb_W0, eb_b0, eb_W1, eb_b1, eb_W2, eb_b2,
           eb_W3, eb_b3, eb_g, eb_beta, nb_W0, nb_b0, nb_W1, nb_b1, nb_W2,
           nb_b2, nb_W3, nb_b3, nb_g, nb_beta):
    n, h = x.shape
    senders = edge_index[0]
    receivers = edge_index[1]

    r2 = lambda v: v.reshape(1, h)

    ts, tr = _make_tables(x, eb_W0[:h], eb_W0[h:2 * h])
    gs = _sc_gather(ts, senders)
    gr = _sc_gather(tr, receivers)
    en, eo = _edge_mlp(gs, gr, edge_attr, eb_W0[2 * h:], r2(eb_b0), eb_W1,
                       r2(eb_b1), eb_W2, r2(eb_b2), eb_W3, r2(eb_b3),
                       r2(eb_g), r2(eb_beta))
    parts, n_pad = _sc_scatter(en, receivers, n)
    xo = _node_mlp(x, parts[:n], parts[n_pad:n_pad + n],
                   nb_W0[:h], nb_W0[h:], r2(nb_b0),
                   nb_W1, r2(nb_b1), nb_W2, r2(nb_b2), nb_W3, r2(nb_b3),
                   r2(nb_g), r2(nb_beta))
    return (xo, eo)


# speedup vs baseline: 2.5478x; 1.5856x over previous
import functools

import jax
import jax.numpy as jnp
from jax import lax
from jax.experimental import pallas as pl
from jax.experimental.pallas import tpu as pltpu
from jax.experimental.pallas import tpu_sc as plsc

_PREC = lax.Precision.DEFAULT


def _tables_body(x_ref, ws_ref, wr_ref, ts_ref, tr_ref):
    xb = x_ref[...]
    ts_ref[...] = jnp.dot(xb, ws_ref[...], preferred_element_type=jnp.float32,
                          precision=_PREC)
    tr_ref[...] = jnp.dot(xb, wr_ref[...], preferred_element_type=jnp.float32,
                          precision=_PREC)


def _make_tables(x, ws, wr):
    n, h = x.shape
    tb = 2000
    return pl.pallas_call(
        _tables_body,
        grid=(n // tb,),
        in_specs=[
            pl.BlockSpec((tb, h), lambda i: (i, 0)),
            pl.BlockSpec((h, h), lambda i: (0, 0)),
            pl.BlockSpec((h, h), lambda i: (0, 0)),
        ],
        out_specs=[
            pl.BlockSpec((tb, h), lambda i: (i, 0)),
            pl.BlockSpec((tb, h), lambda i: (i, 0)),
        ],
        out_shape=[jax.ShapeDtypeStruct((n, h), jnp.float32)] * 2,
    )(x, ws, wr)


_GW = 80


def _sc_gather(table, idx):
    n, h = table.shape
    e = idx.shape[0]
    mesh = plsc.VectorSubcoreMesh(core_axis_name="core",
                                  subcore_axis_name="subcore")

    @functools.partial(
        pl.kernel,
        out_type=jax.ShapeDtypeStruct((e, h), jnp.float32),
        mesh=mesh,
    )
    def k(t_hbm, i_hbm, o_hbm):
        def body(i_vmem, o_vmem):
            pltpu.sync_copy(t_hbm.at[i_vmem.at[0, 0]], o_vmem)

        pltpu.emit_pipeline(
            body,
            grid=(e // _GW,),
            in_specs=[pl.BlockSpec((1, 1, _GW), lambda i: (i, 0, 0))],
            out_specs=[pl.BlockSpec((_GW, h), lambda i: (i, 0))],
            core_axis_name=("core", "subcore"),
            dimension_semantics=(pltpu.PARALLEL,),
        )(i_hbm, o_hbm)

    return k(table, idx.reshape(e // _GW, 1, _GW))


def _sc_scatter(en, receivers, n):
    e, h = en.shape
    n_sub = 16
    n_pad = 10240
    rows_per_sub = n_pad // n_sub
    zb = 128
    mesh = plsc.VectorSubcoreMesh(core_axis_name="core",
                                  subcore_axis_name="subcore")

    @functools.partial(
        pl.kernel,
        out_type=jax.ShapeDtypeStruct((2 * n_pad, h), jnp.float32),
        mesh=mesh,
        scratch_types=[
            pltpu.VMEM((zb, h), jnp.float32),
            pltpu.VMEM_SHARED((n_pad, h), jnp.float32),
        ],
    )
    def k(en_hbm, r_hbm, out_hbm, zbuf, agg_sh):
        cid = lax.axis_index("core")
        sid = lax.axis_index("subcore")

        @pl.loop(0, zb)
        def _(rr):
            for j in range(h // 16):
                zbuf.at[pl.ds(rr, 1), pl.ds(j * 16, 16)][...] = (
                    jnp.zeros((1, 16), jnp.float32))

        @pl.loop(0, rows_per_sub // zb)
        def _(kk):
            pltpu.sync_copy(
                zbuf, agg_sh.at[pl.ds(sid * rows_per_sub + kk * zb, zb)])

        plsc.subcore_barrier()

        def body(en_vmem, r_vmem):
            pltpu.sync_copy(en_vmem, agg_sh.at[r_vmem.at[0, 0]], add=True)

        pltpu.emit_pipeline(
            body,
            grid=(e // _GW,),
            in_specs=[pl.BlockSpec((_GW, h), lambda i: (i, 0)),
                      pl.BlockSpec((1, 1, _GW), lambda i: (i, 0, 0))],
            out_specs=[],
            core_axis_name=("core", "subcore"),
            dimension_semantics=(pltpu.PARALLEL,),
        )(en_hbm, r_hbm)

        plsc.subcore_barrier()

        @pl.loop(0, rows_per_sub // zb)
        def _(kk):
            pltpu.sync_copy(
                agg_sh.at[pl.ds(sid * rows_per_sub + kk * zb, zb)], zbuf)
            pltpu.sync_copy(
                zbuf,
                out_hbm.at[
                    pl.ds(cid * n_pad + sid * rows_per_sub + kk * zb, zb)])

    return k(en, receivers.reshape(e // _GW, 1, _GW)), n_pad


def _edge_body(gs_ref, gr_ref, attr_ref, w0e, b0, w1, b1, w2, b2, w3, b3,
               g, beta, en_ref, eo_ref):
    attr = attr_ref[...]
    h = (gs_ref[...] + gr_ref[...] + b0[...]
         + jnp.dot(attr, w0e[...], preferred_element_type=jnp.float32,
                   precision=_PREC))
    h = jnp.maximum(h, 0.0)
    h = jnp.maximum(
        jnp.dot(h, w1[...], preferred_element_type=jnp.float32,
                precision=_PREC) + b1[...], 0.0)
    h = jnp.maximum(
        jnp.dot(h, w2[...], preferred_element_type=jnp.float32,
                precision=_PREC) + b2[...], 0.0)
    h = jnp.dot(h, w3[...], preferred_element_type=jnp.float32,
                precision=_PREC) + b3[...]
    mu = jnp.mean(h, axis=-1, keepdims=True)
    d = h - mu
    var = jnp.mean(d * d, axis=-1, keepdims=True)
    en = (d * lax.rsqrt(var + 1e-5)) * g[...] + beta[...]
    en_ref[...] = en
    eo_ref[...] = attr + en


def _edge_mlp(gs, gr, attr, w0e, b0, w1, b1, w2, b2, w3, b3, g, beta):
    e, h = attr.shape
    te = 512
    row = lambda i: (i, 0)
    whole = lambda i: (0, 0)
    wspec = pl.BlockSpec((h, h), whole)
    bspec = pl.BlockSpec((1, h), whole)
    return pl.pallas_call(
        _edge_body,
        grid=(e // te,),
        in_specs=[pl.BlockSpec((te, h), row)] * 3
        + [wspec, bspec, wspec, bspec, wspec, bspec, wspec, bspec,
           bspec, bspec],
        out_specs=[pl.BlockSpec((te, h), row)] * 2,
        out_shape=[jax.ShapeDtypeStruct((e, h), jnp.float32)] * 2,
    )(gs, gr, attr, w0e, b0, w1, b1, w2, b2, w3, b3, g, beta)


def _node_body(x_ref, p0_ref, p1_ref, wx, wa, b0, w1, b1, w2, b2, w3, b3,
               g, beta, xo_ref):
    xb = x_ref[...]
    agg = p0_ref[...] + p1_ref[...]
    h = (jnp.dot(xb, wx[...], preferred_element_type=jnp.float32,
                 precision=_PREC)
         + jnp.dot(agg, wa[...], preferred_element_type=jnp.float32,
                   precision=_PREC) + b0[...])
    h = jnp.maximum(h, 0.0)
    h = jnp.maximum(
        jnp.dot(h, w1[...], preferred_element_type=jnp.float32,
                precision=_PREC) + b1[...], 0.0)
    h = jnp.maximum(
        jnp.dot(h, w2[...], preferred_element_type=jnp.float32,
                precision=_PREC) + b2[...], 0.0)
    h = jnp.dot(h, w3[...], preferred_element_type=jnp.float32,
                precision=_PREC) + b3[...]
    mu = jnp.mean(h, axis=-1, keepdims=True)
    d = h - mu
    var = jnp.mean(d * d, axis=-1, keepdims=True)
    xo_ref[...] = xb + (d * lax.rsqrt(var + 1e-5)) * g[...] + beta[...]


def _node_mlp(x, p0, p1, wx, wa, b0, w1, b1, w2, b2, w3, b3, g, beta):
    n, h = x.shape
    tn = 1000
    row = lambda i: (i, 0)
    whole = lambda i: (0, 0)
    wspec = pl.BlockSpec((h, h), whole)
    bspec = pl.BlockSpec((1, h), whole)
    return pl.pallas_call(
        _node_body,
        grid=(n // tn,),
        in_specs=[pl.BlockSpec((tn, h), row)] * 3
        + [wspec, wspec, bspec, wspec, bspec, wspec, bspec, wspec, bspec,
           bspec, bspec],
        out_specs=pl.BlockSpec((tn, h), row),
        out_shape=jax.ShapeDtypeStruct((n, h), jnp.float32),
    )(x, p0, p1, wx, wa, b0, w1, b1, w2, b2, w3, b3, g, beta)


def kernel(x, edge_index, edge_attr, eb_W0, eb_b0, eb_W1, eb_b1, eb_W2, eb_b2,
           eb_W3, eb_b3, eb_g, eb_beta, nb_W0, nb_b0, nb_W1, nb_b1, nb_W2,
           nb_b2, nb_W3, nb_b3, nb_g, nb_beta):
    n, h = x.shape
    senders = edge_index[0]
    receivers = edge_index[1]

    r2 = lambda v: v.reshape(1, h)

    ts, tr = _make_tables(x, eb_W0[:h], eb_W0[h:2 * h])
    gs = _sc_gather(ts, senders)
    gr = _sc_gather(tr, receivers)
    en, eo = _edge_mlp(gs, gr, edge_attr, eb_W0[2 * h:], r2(eb_b0), eb_W1,
                       r2(eb_b1), eb_W2, r2(eb_b2), eb_W3, r2(eb_b3),
                       r2(eb_g), r2(eb_beta))
    parts, n_pad = _sc_scatter(en, receivers, n)
    xo = _node_mlp(x, parts[:n], parts[n_pad:n_pad + n],
                   nb_W0[:h], nb_W0[h:], r2(nb_b0),
                   nb_W1, r2(nb_b1), nb_W2, r2(nb_b2), nb_W3, r2(nb_b3),
                   r2(nb_g), r2(nb_beta))
    return (xo, eo)


# speedup vs baseline: 3.0918x; 1.2135x over previous
import functools

import jax
import jax.numpy as jnp
from jax import lax
from jax.experimental import pallas as pl
from jax.experimental.pallas import tpu as pltpu
from jax.experimental.pallas import tpu_sc as plsc

_PREC = lax.Precision.DEFAULT


def _tables_body(x_ref, ws_ref, wr_ref, ts_ref, tr_ref):
    xb = x_ref[...]
    ts_ref[...] = jnp.dot(xb, ws_ref[...], preferred_element_type=jnp.float32,
                          precision=_PREC)
    tr_ref[...] = jnp.dot(xb, wr_ref[...], preferred_element_type=jnp.float32,
                          precision=_PREC)


def _make_tables(x, ws, wr):
    n, h = x.shape
    tb = 2000
    return pl.pallas_call(
        _tables_body,
        grid=(n // tb,),
        in_specs=[
            pl.BlockSpec((tb, h), lambda i: (i, 0)),
            pl.BlockSpec((h, h), lambda i: (0, 0)),
            pl.BlockSpec((h, h), lambda i: (0, 0)),
        ],
        out_specs=[
            pl.BlockSpec((tb, h), lambda i: (i, 0)),
            pl.BlockSpec((tb, h), lambda i: (i, 0)),
        ],
        out_shape=[jax.ShapeDtypeStruct((n, h), jnp.float32)] * 2,
    )(x, ws, wr)


_GW = 80


def _sc_gather(table, idx):
    n, h = table.shape
    e = idx.shape[0]
    mesh = plsc.VectorSubcoreMesh(core_axis_name="core",
                                  subcore_axis_name="subcore")

    @functools.partial(
        pl.kernel,
        out_type=jax.ShapeDtypeStruct((e, h), jnp.float32),
        mesh=mesh,
    )
    def k(t_hbm, i_hbm, o_hbm):
        def body(i_vmem, o_vmem):
            pltpu.sync_copy(t_hbm.at[i_vmem.at[0, 0]], o_vmem)

        pltpu.emit_pipeline(
            body,
            grid=(e // _GW,),
            in_specs=[pl.BlockSpec((1, 1, _GW), lambda i: (i, 0, 0))],
            out_specs=[pl.BlockSpec((_GW, h), lambda i: (i, 0))],
            core_axis_name=("core", "subcore"),
            dimension_semantics=(pltpu.PARALLEL,),
        )(i_hbm, o_hbm)

    return k(table, idx.reshape(e // _GW, 1, _GW))


def _sc_scatter(en, receivers, n):
    e, h = en.shape
    n_sub = 16
    n_pad = 10240
    rows_per_sub = n_pad // n_sub
    zb = 128
    mesh = plsc.VectorSubcoreMesh(core_axis_name="core",
                                  subcore_axis_name="subcore")

    @functools.partial(
        pl.kernel,
        out_type=jax.ShapeDtypeStruct((2 * n_pad, h), jnp.float32),
        mesh=mesh,
        scratch_types=[
            pltpu.VMEM((zb, h), jnp.float32),
            pltpu.VMEM_SHARED((n_pad, h), jnp.float32),
        ],
    )
    def k(en_hbm, r_hbm, out_hbm, zbuf, agg_sh):
        cid = lax.axis_index("core")
        sid = lax.axis_index("subcore")

        @pl.loop(0, zb)
        def _(rr):
            for j in range(h // 16):
                zbuf.at[pl.ds(rr, 1), pl.ds(j * 16, 16)][...] = (
                    jnp.zeros((1, 16), jnp.float32))

        @pl.loop(0, rows_per_sub // zb)
        def _(kk):
            pltpu.sync_copy(
                zbuf, agg_sh.at[pl.ds(sid * rows_per_sub + kk * zb, zb)])

        plsc.subcore_barrier()

        def body(en_vmem, r_vmem):
            pltpu.sync_copy(en_vmem, agg_sh.at[r_vmem.at[0, 0]], add=True)

        pltpu.emit_pipeline(
            body,
            grid=(e // _GW,),
            in_specs=[pl.BlockSpec((_GW, h), lambda i: (i, 0)),
                      pl.BlockSpec((1, 1, _GW), lambda i: (i, 0, 0))],
            out_specs=[],
            core_axis_name=("core", "subcore"),
            dimension_semantics=(pltpu.PARALLEL,),
        )(en_hbm, r_hbm)

        plsc.subcore_barrier()

        @pl.loop(0, rows_per_sub // zb)
        def _(kk):
            pltpu.sync_copy(
                agg_sh.at[pl.ds(sid * rows_per_sub + kk * zb, zb)], zbuf)
            pltpu.sync_copy(
                zbuf,
                out_hbm.at[
                    pl.ds(cid * n_pad + sid * rows_per_sub + kk * zb, zb)])

    return k(en, receivers.reshape(e // _GW, 1, _GW)), n_pad


def _edge_body(gs_ref, gr_ref, attr_ref, w0e, b0, w1, b1, w2, b2, w3, b3,
               g, beta, en_ref, eo_ref):
    attr = attr_ref[...]
    h = (gs_ref[...] + gr_ref[...] + b0[...]
         + jnp.dot(attr, w0e[...], preferred_element_type=jnp.float32,
                   precision=_PREC))
    h = jnp.maximum(h, 0.0)
    h = jnp.maximum(
        jnp.dot(h, w1[...], preferred_element_type=jnp.float32,
                precision=_PREC) + b1[...], 0.0)
    h = jnp.maximum(
        jnp.dot(h, w2[...], preferred_element_type=jnp.float32,
                precision=_PREC) + b2[...], 0.0)
    h = jnp.dot(h, w3[...], preferred_element_type=jnp.float32,
                precision=_PREC) + b3[...]
    mu = jnp.mean(h, axis=-1, keepdims=True)
    d = h - mu
    var = jnp.mean(d * d, axis=-1, keepdims=True)
    en = (d * lax.rsqrt(var + 1e-5)) * g[...] + beta[...]
    en_ref[...] = en
    eo_ref[...] = attr + en


def _edge_mlp(gs, gr, attr, w0e, b0, w1, b1, w2, b2, w3, b3, g, beta):
    e, h = attr.shape
    te = 1000
    row = lambda i: (i, 0)
    whole = lambda i: (0, 0)
    wspec = pl.BlockSpec((h, h), whole)
    bspec = pl.BlockSpec((1, h), whole)
    return pl.pallas_call(
        _edge_body,
        grid=(e // te,),
        in_specs=[pl.BlockSpec((te, h), row)] * 3
        + [wspec, bspec, wspec, bspec, wspec, bspec, wspec, bspec,
           bspec, bspec],
        out_specs=[pl.BlockSpec((te, h), row)] * 2,
        out_shape=[jax.ShapeDtypeStruct((e, h), jnp.float32)] * 2,
    )(gs, gr, attr, w0e, b0, w1, b1, w2, b2, w3, b3, g, beta)


def _node_body(x_ref, p0_ref, p1_ref, wx, wa, b0, w1, b1, w2, b2, w3, b3,
               g, beta, xo_ref):
    xb = x_ref[...]
    agg = p0_ref[...] + p1_ref[...]
    h = (jnp.dot(xb, wx[...], preferred_element_type=jnp.float32,
                 precision=_PREC)
         + jnp.dot(agg, wa[...], preferred_element_type=jnp.float32,
                   precision=_PREC) + b0[...])
    h = jnp.maximum(h, 0.0)
    h = jnp.maximum(
        jnp.dot(h, w1[...], preferred_element_type=jnp.float32,
                precision=_PREC) + b1[...], 0.0)
    h = jnp.maximum(
        jnp.dot(h, w2[...], preferred_element_type=jnp.float32,
                precision=_PREC) + b2[...], 0.0)
    h = jnp.dot(h, w3[...], preferred_element_type=jnp.float32,
                precision=_PREC) + b3[...]
    mu = jnp.mean(h, axis=-1, keepdims=True)
    d = h - mu
    var = jnp.mean(d * d, axis=-1, keepdims=True)
    xo_ref[...] = xb + (d * lax.rsqrt(var + 1e-5)) * g[...] + beta[...]


def _node_mlp(x, p0, p1, wx, wa, b0, w1, b1, w2, b2, w3, b3, g, beta):
    n, h = x.shape
    tn = 1000
    row = lambda i: (i, 0)
    whole = lambda i: (0, 0)
    wspec = pl.BlockSpec((h, h), whole)
    bspec = pl.BlockSpec((1, h), whole)
    return pl.pallas_call(
        _node_body,
        grid=(n // tn,),
        in_specs=[pl.BlockSpec((tn, h), row)] * 3
        + [wspec, wspec, bspec, wspec, bspec, wspec, bspec, wspec, bspec,
           bspec, bspec],
        out_specs=pl.BlockSpec((tn, h), row),
        out_shape=jax.ShapeDtypeStruct((n, h), jnp.float32),
    )(x, p0, p1, wx, wa, b0, w1, b1, w2, b2, w3, b3, g, beta)


def kernel(x, edge_index, edge_attr, eb_W0, eb_b0, eb_W1, eb_b1, eb_W2, eb_b2,
           eb_W3, eb_b3, eb_g, eb_beta, nb_W0, nb_b0, nb_W1, nb_b1, nb_W2,
           nb_b2, nb_W3, nb_b3, nb_g, nb_beta):
    n, h = x.shape
    senders = edge_index[0]
    receivers = edge_index[1]

    r2 = lambda v: v.reshape(1, h)

    ts, tr = _make_tables(x, eb_W0[:h], eb_W0[h:2 * h])
    gs = _sc_gather(ts, senders)
    gr = _sc_gather(tr, receivers)
    en, eo = _edge_mlp(gs, gr, edge_attr, eb_W0[2 * h:], r2(eb_b0), eb_W1,
                       r2(eb_b1), eb_W2, r2(eb_b2), eb_W3, r2(eb_b3),
                       r2(eb_g), r2(eb_beta))
    parts, n_pad = _sc_scatter(en, receivers, n)
    xo = _node_mlp(x, parts[:n], parts[n_pad:n_pad + n],
                   nb_W0[:h], nb_W0[h:], r2(nb_b0),
                   nb_W1, r2(nb_b1), nb_W2, r2(nb_b2), nb_W3, r2(nb_b3),
                   r2(nb_g), r2(nb_beta))
    return (xo, eo)


# speedup vs baseline: 3.6189x; 1.1705x over previous
import functools

import jax
import jax.numpy as jnp
from jax import lax
from jax.experimental import pallas as pl
from jax.experimental.pallas import tpu as pltpu
from jax.experimental.pallas import tpu_sc as plsc

_PREC = lax.Precision.DEFAULT


def _tables_body(x_ref, ws_ref, wr_ref, ts_ref, tr_ref):
    xb = x_ref[...]
    ts_ref[...] = jnp.dot(xb, ws_ref[...], preferred_element_type=jnp.float32,
                          precision=_PREC)
    tr_ref[...] = jnp.dot(xb, wr_ref[...], preferred_element_type=jnp.float32,
                          precision=_PREC)


def _make_tables(x, ws, wr):
    n, h = x.shape
    tb = 2000
    return pl.pallas_call(
        _tables_body,
        grid=(n // tb,),
        in_specs=[
            pl.BlockSpec((tb, h), lambda i: (i, 0)),
            pl.BlockSpec((h, h), lambda i: (0, 0)),
            pl.BlockSpec((h, h), lambda i: (0, 0)),
        ],
        out_specs=[
            pl.BlockSpec((tb, h), lambda i: (i, 0)),
            pl.BlockSpec((tb, h), lambda i: (i, 0)),
        ],
        out_shape=[jax.ShapeDtypeStruct((n, h), jnp.float32)] * 2,
    )(x, ws, wr)


_GW = 80


def _sc_gather(table, idx):
    n, h = table.shape
    e = idx.shape[0]
    mesh = plsc.VectorSubcoreMesh(core_axis_name="core",
                                  subcore_axis_name="subcore")

    @functools.partial(
        pl.kernel,
        out_type=jax.ShapeDtypeStruct((e, h), jnp.float32),
        mesh=mesh,
    )
    def k(t_hbm, i_hbm, o_hbm):
        def body(i_vmem, o_vmem):
            pltpu.sync_copy(t_hbm.at[i_vmem.at[0, 0]], o_vmem)

        pltpu.emit_pipeline(
            body,
            grid=(e // _GW,),
            in_specs=[pl.BlockSpec((1, 1, _GW), lambda i: (i, 0, 0))],
            out_specs=[pl.BlockSpec((_GW, h), lambda i: (i, 0))],
            core_axis_name=("core", "subcore"),
            dimension_semantics=(pltpu.PARALLEL,),
        )(i_hbm, o_hbm)

    return k(table, idx.reshape(e // _GW, 1, _GW))


def _sc_scatter(en, receivers, n):
    e, h = en.shape
    n_sub = 16
    n_pad = 10240
    rows_per_sub = n_pad // n_sub
    zb = 128
    mesh = plsc.VectorSubcoreMesh(core_axis_name="core",
                                  subcore_axis_name="subcore")

    @functools.partial(
        pl.kernel,
        out_type=jax.ShapeDtypeStruct((2 * n_pad, h), jnp.float32),
        mesh=mesh,
        scratch_types=[
            pltpu.VMEM((zb, h), jnp.float32),
            pltpu.VMEM_SHARED((n_pad, h), jnp.float32),
        ],
    )
    def k(en_hbm, r_hbm, out_hbm, zbuf, agg_sh):
        cid = lax.axis_index("core")
        sid = lax.axis_index("subcore")

        @pl.loop(0, zb)
        def _(rr):
            for j in range(h // 16):
                zbuf.at[pl.ds(rr, 1), pl.ds(j * 16, 16)][...] = (
                    jnp.zeros((1, 16), jnp.float32))

        @pl.loop(0, rows_per_sub // zb)
        def _(kk):
            pltpu.sync_copy(
                zbuf, agg_sh.at[pl.ds(sid * rows_per_sub + kk * zb, zb)])

        plsc.subcore_barrier()

        def body(en_vmem, r_vmem):
            pltpu.sync_copy(en_vmem, agg_sh.at[r_vmem.at[0, 0]], add=True)

        pltpu.emit_pipeline(
            body,
            grid=(e // _GW,),
            in_specs=[pl.BlockSpec((_GW, h), lambda i: (i, 0)),
                      pl.BlockSpec((1, 1, _GW), lambda i: (i, 0, 0))],
            out_specs=[],
            core_axis_name=("core", "subcore"),
            dimension_semantics=(pltpu.PARALLEL,),
        )(en_hbm, r_hbm)

        plsc.subcore_barrier()

        @pl.loop(0, rows_per_sub // zb)
        def _(kk):
            pltpu.sync_copy(
                agg_sh.at[pl.ds(sid * rows_per_sub + kk * zb, zb)], zbuf)
            pltpu.sync_copy(
                zbuf,
                out_hbm.at[
                    pl.ds(cid * n_pad + sid * rows_per_sub + kk * zb, zb)])

    return k(en, receivers.reshape(e // _GW, 1, _GW)), n_pad


def _edge_body(gs_ref, gr_ref, attr_ref, w0e, b0, w1, b1, w2, b2, w3, b3,
               g, beta, en_ref, eo_ref):
    attr = attr_ref[...]
    h = (gs_ref[...] + gr_ref[...] + b0[...]
         + jnp.dot(attr, w0e[...], preferred_element_type=jnp.float32,
                   precision=_PREC))
    h = jnp.maximum(h, 0.0)
    h = jnp.maximum(
        jnp.dot(h, w1[...], preferred_element_type=jnp.float32,
                precision=_PREC) + b1[...], 0.0)
    h = jnp.maximum(
        jnp.dot(h, w2[...], preferred_element_type=jnp.float32,
                precision=_PREC) + b2[...], 0.0)
    h = jnp.dot(h, w3[...], preferred_element_type=jnp.float32,
                precision=_PREC) + b3[...]
    mu = jnp.mean(h, axis=-1, keepdims=True)
    d = h - mu
    var = jnp.mean(d * d, axis=-1, keepdims=True)
    en = (d * lax.rsqrt(var + 1e-5)) * g[...] + beta[...]
    en_ref[...] = en
    eo_ref[...] = attr + en


def _edge_mlp(gs, gr, attr, w0e, b0, w1, b1, w2, b2, w3, b3, g, beta):
    e, h = attr.shape
    te = 2000
    row = lambda i: (i, 0)
    whole = lambda i: (0, 0)
    wspec = pl.BlockSpec((h, h), whole)
    bspec = pl.BlockSpec((1, h), whole)
    return pl.pallas_call(
        _edge_body,
        grid=(e // te,),
        in_specs=[pl.BlockSpec((te, h), row)] * 3
        + [wspec, bspec, wspec, bspec, wspec, bspec, wspec, bspec,
           bspec, bspec],
        out_specs=[pl.BlockSpec((te, h), row)] * 2,
        out_shape=[jax.ShapeDtypeStruct((e, h), jnp.float32)] * 2,
    )(gs, gr, attr, w0e, b0, w1, b1, w2, b2, w3, b3, g, beta)


def _node_body(x_ref, p0_ref, p1_ref, wx, wa, b0, w1, b1, w2, b2, w3, b3,
               g, beta, xo_ref):
    xb = x_ref[...]
    agg = p0_ref[...] + p1_ref[...]
    h = (jnp.dot(xb, wx[...], preferred_element_type=jnp.float32,
                 precision=_PREC)
         + jnp.dot(agg, wa[...], preferred_element_type=jnp.float32,
                   precision=_PREC) + b0[...])
    h = jnp.maximum(h, 0.0)
    h = jnp.maximum(
        jnp.dot(h, w1[...], preferred_element_type=jnp.float32,
                precision=_PREC) + b1[...], 0.0)
    h = jnp.maximum(
        jnp.dot(h, w2[...], preferred_element_type=jnp.float32,
                precision=_PREC) + b2[...], 0.0)
    h = jnp.dot(h, w3[...], preferred_element_type=jnp.float32,
                precision=_PREC) + b3[...]
    mu = jnp.mean(h, axis=-1, keepdims=True)
    d = h - mu
    var = jnp.mean(d * d, axis=-1, keepdims=True)
    xo_ref[...] = xb + (d * lax.rsqrt(var + 1e-5)) * g[...] + beta[...]


def _node_mlp(x, p0, p1, wx, wa, b0, w1, b1, w2, b2, w3, b3, g, beta):
    n, h = x.shape
    tn = 1000
    row = lambda i: (i, 0)
    whole = lambda i: (0, 0)
    wspec = pl.BlockSpec((h, h), whole)
    bspec = pl.BlockSpec((1, h), whole)
    return pl.pallas_call(
        _node_body,
        grid=(n // tn,),
        in_specs=[pl.BlockSpec((tn, h), row)] * 3
        + [wspec, wspec, bspec, wspec, bspec, wspec, bspec, wspec, bspec,
           bspec, bspec],
        out_specs=pl.BlockSpec((tn, h), row),
        out_shape=jax.ShapeDtypeStruct((n, h), jnp.float32),
    )(x, p0, p1, wx, wa, b0, w1, b1, w2, b2, w3, b3, g, beta)


def kernel(x, edge_index, edge_attr, eb_W0, eb_b0, eb_W1, eb_b1, eb_W2, eb_b2,
           eb_W3, eb_b3, eb_g, eb_beta, nb_W0, nb_b0, nb_W1, nb_b1, nb_W2,
           nb_b2, nb_W3, nb_b3, nb_g, nb_beta):
    n, h = x.shape
    senders = edge_index[0]
    receivers = edge_index[1]

    r2 = lambda v: v.reshape(1, h)

    ts, tr = _make_tables(x, eb_W0[:h], eb_W0[h:2 * h])
    gs = _sc_gather(ts, senders)
    gr = _sc_gather(tr, receivers)
    en, eo = _edge_mlp(gs, gr, edge_attr, eb_W0[2 * h:], r2(eb_b0), eb_W1,
                       r2(eb_b1), eb_W2, r2(eb_b2), eb_W3, r2(eb_b3),
                       r2(eb_g), r2(eb_beta))
    parts, n_pad = _sc_scatter(en, receivers, n)
    xo = _node_mlp(x, parts[:n], parts[n_pad:n_pad + n],
                   nb_W0[:h], nb_W0[h:], r2(nb_b0),
                   nb_W1, r2(nb_b1), nb_W2, r2(nb_b2), nb_W3, r2(nb_b3),
                   r2(nb_g), r2(nb_beta))
    return (xo, eo)


# speedup vs baseline: 3.8817x; 1.0726x over previous
import functools

import jax
import jax.numpy as jnp
from jax import lax
from jax.experimental import pallas as pl
from jax.experimental.pallas import tpu as pltpu
from jax.experimental.pallas import tpu_sc as plsc

_PREC = lax.Precision.DEFAULT


def _tables_body(x_ref, ws_ref, wr_ref, ts_ref, tr_ref):
    xb = x_ref[...]
    ts_ref[...] = jnp.dot(xb, ws_ref[...], preferred_element_type=jnp.float32,
                          precision=_PREC)
    tr_ref[...] = jnp.dot(xb, wr_ref[...], preferred_element_type=jnp.float32,
                          precision=_PREC)


def _make_tables(x, ws, wr):
    n, h = x.shape
    tb = 2000
    return pl.pallas_call(
        _tables_body,
        grid=(n // tb,),
        in_specs=[
            pl.BlockSpec((tb, h), lambda i: (i, 0)),
            pl.BlockSpec((h, h), lambda i: (0, 0)),
            pl.BlockSpec((h, h), lambda i: (0, 0)),
        ],
        out_specs=[
            pl.BlockSpec((tb, h), lambda i: (i, 0)),
            pl.BlockSpec((tb, h), lambda i: (i, 0)),
        ],
        out_shape=[jax.ShapeDtypeStruct((n, h), jnp.float32)] * 2,
    )(x, ws, wr)


_GW = 80


def _sc_gather(table, idx):
    n, h = table.shape
    e = idx.shape[0]
    mesh = plsc.VectorSubcoreMesh(core_axis_name="core",
                                  subcore_axis_name="subcore")

    @functools.partial(
        pl.kernel,
        out_type=jax.ShapeDtypeStruct((e, h), jnp.float32),
        mesh=mesh,
    )
    def k(t_hbm, i_hbm, o_hbm):
        def body(i_vmem, o_vmem):
            pltpu.sync_copy(t_hbm.at[i_vmem.at[0, 0]], o_vmem)

        pltpu.emit_pipeline(
            body,
            grid=(e // _GW,),
            in_specs=[pl.BlockSpec((1, 1, _GW), lambda i: (i, 0, 0))],
            out_specs=[pl.BlockSpec((_GW, h), lambda i: (i, 0))],
            core_axis_name=("core", "subcore"),
            dimension_semantics=(pltpu.PARALLEL,),
        )(i_hbm, o_hbm)

    return k(table, idx.reshape(e // _GW, 1, _GW))


def _sc_scatter(en, receivers, n):
    e, h = en.shape
    n_sub = 16
    n_pad = 10240
    rows_per_sub = n_pad // n_sub
    zb = 128
    mesh = plsc.VectorSubcoreMesh(core_axis_name="core",
                                  subcore_axis_name="subcore")

    @functools.partial(
        pl.kernel,
        out_type=jax.ShapeDtypeStruct((2 * n_pad, h), jnp.float32),
        mesh=mesh,
        scratch_types=[
            pltpu.VMEM((zb, h), jnp.float32),
            pltpu.VMEM_SHARED((n_pad, h), jnp.float32),
        ],
    )
    def k(en_hbm, r_hbm, out_hbm, zbuf, agg_sh):
        cid = lax.axis_index("core")
        sid = lax.axis_index("subcore")

        @pl.loop(0, zb)
        def _(rr):
            for j in range(h // 16):
                zbuf.at[pl.ds(rr, 1), pl.ds(j * 16, 16)][...] = (
                    jnp.zeros((1, 16), jnp.float32))

        @pl.loop(0, rows_per_sub // zb)
        def _(kk):
            pltpu.sync_copy(
                zbuf, agg_sh.at[pl.ds(sid * rows_per_sub + kk * zb, zb)])

        plsc.subcore_barrier()

        def body(en_vmem, r_vmem):
            pltpu.sync_copy(en_vmem, agg_sh.at[r_vmem.at[0, 0]], add=True)

        pltpu.emit_pipeline(
            body,
            grid=(e // _GW,),
            in_specs=[pl.BlockSpec((_GW, h), lambda i: (i, 0)),
                      pl.BlockSpec((1, 1, _GW), lambda i: (i, 0, 0))],
            out_specs=[],
            core_axis_name=("core", "subcore"),
            dimension_semantics=(pltpu.PARALLEL,),
        )(en_hbm, r_hbm)

        plsc.subcore_barrier()

        @pl.loop(0, rows_per_sub // zb)
        def _(kk):
            pltpu.sync_copy(
                agg_sh.at[pl.ds(sid * rows_per_sub + kk * zb, zb)], zbuf)
            pltpu.sync_copy(
                zbuf,
                out_hbm.at[
                    pl.ds(cid * n_pad + sid * rows_per_sub + kk * zb, zb)])

    return k(en, receivers.reshape(e // _GW, 1, _GW)), n_pad


def _edge_body(gs_ref, gr_ref, attr_ref, w0e, b0, w1, b1, w2, b2, w3, b3,
               g, beta, en_ref, eo_ref):
    attr = attr_ref[...]
    h = (gs_ref[...] + gr_ref[...] + b0[...]
         + jnp.dot(attr, w0e[...], preferred_element_type=jnp.float32,
                   precision=_PREC))
    h = jnp.maximum(h, 0.0)
    h = jnp.maximum(
        jnp.dot(h, w1[...], preferred_element_type=jnp.float32,
                precision=_PREC) + b1[...], 0.0)
    h = jnp.maximum(
        jnp.dot(h, w2[...], preferred_element_type=jnp.float32,
                precision=_PREC) + b2[...], 0.0)
    h = jnp.dot(h, w3[...], preferred_element_type=jnp.float32,
                precision=_PREC) + b3[...]
    mu = jnp.mean(h, axis=-1, keepdims=True)
    d = h - mu
    var = jnp.mean(d * d, axis=-1, keepdims=True)
    en = (d * lax.rsqrt(var + 1e-5)) * g[...] + beta[...]
    en_ref[...] = en
    eo_ref[...] = attr + en


def _edge_mlp(gs, gr, attr, w0e, b0, w1, b1, w2, b2, w3, b3, g, beta):
    e, h = attr.shape
    te = 4000
    row = lambda i: (i, 0)
    whole = lambda i: (0, 0)
    wspec = pl.BlockSpec((h, h), whole)
    bspec = pl.BlockSpec((1, h), whole)
    return pl.pallas_call(
        _edge_body,
        grid=(e // te,),
        in_specs=[pl.BlockSpec((te, h), row)] * 3
        + [wspec, bspec, wspec, bspec, wspec, bspec, wspec, bspec,
           bspec, bspec],
        out_specs=[pl.BlockSpec((te, h), row)] * 2,
        out_shape=[jax.ShapeDtypeStruct((e, h), jnp.float32)] * 2,
    )(gs, gr, attr, w0e, b0, w1, b1, w2, b2, w3, b3, g, beta)


def _node_body(x_ref, p0_ref, p1_ref, wx, wa, b0, w1, b1, w2, b2, w3, b3,
               g, beta, xo_ref):
    xb = x_ref[...]
    agg = p0_ref[...] + p1_ref[...]
    h = (jnp.dot(xb, wx[...], preferred_element_type=jnp.float32,
                 precision=_PREC)
         + jnp.dot(agg, wa[...], preferred_element_type=jnp.float32,
                   precision=_PREC) + b0[...])
    h = jnp.maximum(h, 0.0)
    h = jnp.maximum(
        jnp.dot(h, w1[...], preferred_element_type=jnp.float32,
                precision=_PREC) + b1[...], 0.0)
    h = jnp.maximum(
        jnp.dot(h, w2[...], preferred_element_type=jnp.float32,
                precision=_PREC) + b2[...], 0.0)
    h = jnp.dot(h, w3[...], preferred_element_type=jnp.float32,
                precision=_PREC) + b3[...]
    mu = jnp.mean(h, axis=-1, keepdims=True)
    d = h - mu
    var = jnp.mean(d * d, axis=-1, keepdims=True)
    xo_ref[...] = xb + (d * lax.rsqrt(var + 1e-5)) * g[...] + beta[...]


def _node_mlp(x, p0, p1, wx, wa, b0, w1, b1, w2, b2, w3, b3, g, beta):
    n, h = x.shape
    tn = 2000
    row = lambda i: (i, 0)
    whole = lambda i: (0, 0)
    wspec = pl.BlockSpec((h, h), whole)
    bspec = pl.BlockSpec((1, h), whole)
    return pl.pallas_call(
        _node_body,
        grid=(n // tn,),
        in_specs=[pl.BlockSpec((tn, h), row)] * 3
        + [wspec, wspec, bspec, wspec, bspec, wspec, bspec, wspec, bspec,
           bspec, bspec],
        out_specs=pl.BlockSpec((tn, h), row),
        out_shape=jax.ShapeDtypeStruct((n, h), jnp.float32),
    )(x, p0, p1, wx, wa, b0, w1, b1, w2, b2, w3, b3, g, beta)


def kernel(x, edge_index, edge_attr, eb_W0, eb_b0, eb_W1, eb_b1, eb_W2, eb_b2,
           eb_W3, eb_b3, eb_g, eb_beta, nb_W0, nb_b0, nb_W1, nb_b1, nb_W2,
           nb_b2, nb_W3, nb_b3, nb_g, nb_beta):
    n, h = x.shape
    senders = edge_index[0]
    receivers = edge_index[1]

    r2 = lambda v: v.reshape(1, h)

    ts, tr = _make_tables(x, eb_W0[:h], eb_W0[h:2 * h])
    gs = _sc_gather(ts, senders)
    gr = _sc_gather(tr, receivers)
    en, eo = _edge_mlp(gs, gr, edge_attr, eb_W0[2 * h:], r2(eb_b0), eb_W1,
                       r2(eb_b1), eb_W2, r2(eb_b2), eb_W3, r2(eb_b3),
                       r2(eb_g), r2(eb_beta))
    parts, n_pad = _sc_scatter(en, receivers, n)
    xo = _node_mlp(x, parts[:n], parts[n_pad:n_pad + n],
                   nb_W0[:h], nb_W0[h:], r2(nb_b0),
                   nb_W1, r2(nb_b1), nb_W2, r2(nb_b2), nb_W3, r2(nb_b3),
                   r2(nb_g), r2(nb_beta))
    return (xo, eo)


# speedup vs baseline: 3.9004x; 1.0048x over previous
import functools

import jax
import jax.numpy as jnp
from jax import lax
from jax.experimental import pallas as pl
from jax.experimental.pallas import tpu as pltpu
from jax.experimental.pallas import tpu_sc as plsc

_PREC = lax.Precision.DEFAULT


def _tables_body(x_ref, ws_ref, wr_ref, ts_ref, tr_ref):
    xb = x_ref[...]
    ts_ref[...] = jnp.dot(xb, ws_ref[...], preferred_element_type=jnp.float32,
                          precision=_PREC)
    tr_ref[...] = jnp.dot(xb, wr_ref[...], preferred_element_type=jnp.float32,
                          precision=_PREC)


def _make_tables(x, ws, wr):
    n, h = x.shape
    tb = 2000
    return pl.pallas_call(
        _tables_body,
        grid=(n // tb,),
        in_specs=[
            pl.BlockSpec((tb, h), lambda i: (i, 0)),
            pl.BlockSpec((h, h), lambda i: (0, 0)),
            pl.BlockSpec((h, h), lambda i: (0, 0)),
        ],
        out_specs=[
            pl.BlockSpec((tb, h), lambda i: (i, 0)),
            pl.BlockSpec((tb, h), lambda i: (i, 0)),
        ],
        out_shape=[jax.ShapeDtypeStruct((n, h), jnp.float32)] * 2,
    )(x, ws, wr)


_GW = 80


def _sc_gather(table, idx):
    n, h = table.shape
    e = idx.shape[0]
    mesh = plsc.VectorSubcoreMesh(core_axis_name="core",
                                  subcore_axis_name="subcore")

    @functools.partial(
        pl.kernel,
        out_type=jax.ShapeDtypeStruct((e, h), jnp.float32),
        mesh=mesh,
    )
    def k(t_hbm, i_hbm, o_hbm):
        def body(i_vmem, o_vmem):
            pltpu.sync_copy(t_hbm.at[i_vmem.at[0, 0]], o_vmem)

        pltpu.emit_pipeline(
            body,
            grid=(e // _GW,),
            in_specs=[pl.BlockSpec((1, 1, _GW), lambda i: (i, 0, 0))],
            out_specs=[pl.BlockSpec((_GW, h), lambda i: (i, 0))],
            core_axis_name=("core", "subcore"),
            dimension_semantics=(pltpu.PARALLEL,),
        )(i_hbm, o_hbm)

    return k(table, idx.reshape(e // _GW, 1, _GW))


def _sc_scatter(en, receivers, n):
    e, h = en.shape
    n_sub = 16
    n_pad = 10240
    rows_per_sub = n_pad // n_sub
    zb = 128
    mesh = plsc.VectorSubcoreMesh(core_axis_name="core",
                                  subcore_axis_name="subcore")

    @functools.partial(
        pl.kernel,
        out_type=jax.ShapeDtypeStruct((2 * n_pad, h), jnp.float32),
        mesh=mesh,
        scratch_types=[
            pltpu.VMEM((zb, h), jnp.float32),
            pltpu.VMEM_SHARED((n_pad, h), jnp.float32),
        ],
    )
    def k(en_hbm, r_hbm, out_hbm, zbuf, agg_sh):
        cid = lax.axis_index("core")
        sid = lax.axis_index("subcore")

        @pl.loop(0, zb)
        def _(rr):
            for j in range(h // 16):
                zbuf.at[pl.ds(rr, 1), pl.ds(j * 16, 16)][...] = (
                    jnp.zeros((1, 16), jnp.float32))

        @pl.loop(0, rows_per_sub // zb)
        def _(kk):
            pltpu.sync_copy(
                zbuf, agg_sh.at[pl.ds(sid * rows_per_sub + kk * zb, zb)])

        plsc.subcore_barrier()

        def body(en_vmem, r_vmem):
            pltpu.sync_copy(en_vmem, agg_sh.at[r_vmem.at[0, 0]], add=True)

        pltpu.emit_pipeline(
            body,
            grid=(e // _GW,),
            in_specs=[pl.BlockSpec((_GW, h), lambda i: (i, 0)),
                      pl.BlockSpec((1, 1, _GW), lambda i: (i, 0, 0))],
            out_specs=[],
            core_axis_name=("core", "subcore"),
            dimension_semantics=(pltpu.PARALLEL,),
        )(en_hbm, r_hbm)

        plsc.subcore_barrier()

        @pl.loop(0, rows_per_sub // zb)
        def _(kk):
            pltpu.sync_copy(
                agg_sh.at[pl.ds(sid * rows_per_sub + kk * zb, zb)], zbuf)
            pltpu.sync_copy(
                zbuf,
                out_hbm.at[
                    pl.ds(cid * n_pad + sid * rows_per_sub + kk * zb, zb)])

    return k(en, receivers.reshape(e // _GW, 1, _GW)), n_pad


def _edge_body(gs_ref, gr_ref, attr_ref, w0e, b0, w1, b1, w2, b2, w3, b3,
               g, beta, en_ref, eo_ref):
    attr = attr_ref[...]
    h = (gs_ref[...] + gr_ref[...] + b0[...]
         + jnp.dot(attr, w0e[...], preferred_element_type=jnp.float32,
                   precision=_PREC))
    h = jnp.maximum(h, 0.0)
    h = jnp.maximum(
        jnp.dot(h, w1[...], preferred_element_type=jnp.float32,
                precision=_PREC) + b1[...], 0.0)
    h = jnp.maximum(
        jnp.dot(h, w2[...], preferred_element_type=jnp.float32,
                precision=_PREC) + b2[...], 0.0)
    h = jnp.dot(h, w3[...], preferred_element_type=jnp.float32,
                precision=_PREC) + b3[...]
    mu = jnp.mean(h, axis=-1, keepdims=True)
    d = h - mu
    var = jnp.mean(d * d, axis=-1, keepdims=True)
    en = (d * lax.rsqrt(var + 1e-5)) * g[...] + beta[...]
    en_ref[...] = en
    eo_ref[...] = attr + en


def _edge_mlp(gs, gr, attr, w0e, b0, w1, b1, w2, b2, w3, b3, g, beta):
    e, h = attr.shape
    te = 8000
    row = lambda i: (i, 0)
    whole = lambda i: (0, 0)
    wspec = pl.BlockSpec((h, h), whole)
    bspec = pl.BlockSpec((1, h), whole)
    return pl.pallas_call(
        _edge_body,
        grid=(e // te,),
        in_specs=[pl.BlockSpec((te, h), row)] * 3
        + [wspec, bspec, wspec, bspec, wspec, bspec, wspec, bspec,
           bspec, bspec],
        out_specs=[pl.BlockSpec((te, h), row)] * 2,
        out_shape=[jax.ShapeDtypeStruct((e, h), jnp.float32)] * 2,
    )(gs, gr, attr, w0e, b0, w1, b1, w2, b2, w3, b3, g, beta)


def _node_body(x_ref, p0_ref, p1_ref, wx, wa, b0, w1, b1, w2, b2, w3, b3,
               g, beta, xo_ref):
    xb = x_ref[...]
    agg = p0_ref[...] + p1_ref[...]
    h = (jnp.dot(xb, wx[...], preferred_element_type=jnp.float32,
                 precision=_PREC)
         + jnp.dot(agg, wa[...], preferred_element_type=jnp.float32,
                   precision=_PREC) + b0[...])
    h = jnp.maximum(h, 0.0)
    h = jnp.maximum(
        jnp.dot(h, w1[...], preferred_element_type=jnp.float32,
                precision=_PREC) + b1[...], 0.0)
    h = jnp.maximum(
        jnp.dot(h, w2[...], preferred_element_type=jnp.float32,
                precision=_PREC) + b2[...], 0.0)
    h = jnp.dot(h, w3[...], preferred_element_type=jnp.float32,
                precision=_PREC) + b3[...]
    mu = jnp.mean(h, axis=-1, keepdims=True)
    d = h - mu
    var = jnp.mean(d * d, axis=-1, keepdims=True)
    xo_ref[...] = xb + (d * lax.rsqrt(var + 1e-5)) * g[...] + beta[...]


def _node_mlp(x, p0, p1, wx, wa, b0, w1, b1, w2, b2, w3, b3, g, beta):
    n, h = x.shape
    tn = 2000
    row = lambda i: (i, 0)
    whole = lambda i: (0, 0)
    wspec = pl.BlockSpec((h, h), whole)
    bspec = pl.BlockSpec((1, h), whole)
    return pl.pallas_call(
        _node_body,
        grid=(n // tn,),
        in_specs=[pl.BlockSpec((tn, h), row)] * 3
        + [wspec, wspec, bspec, wspec, bspec, wspec, bspec, wspec, bspec,
           bspec, bspec],
        out_specs=pl.BlockSpec((tn, h), row),
        out_shape=jax.ShapeDtypeStruct((n, h), jnp.float32),
    )(x, p0, p1, wx, wa, b0, w1, b1, w2, b2, w3, b3, g, beta)


def kernel(x, edge_index, edge_attr, eb_W0, eb_b0, eb_W1, eb_b1, eb_W2, eb_b2,
           eb_W3, eb_b3, eb_g, eb_beta, nb_W0, nb_b0, nb_W1, nb_b1, nb_W2,
           nb_b2, nb_W3, nb_b3, nb_g, nb_beta):
    n, h = x.shape
    senders = edge_index[0]
    receivers = edge_index[1]

    r2 = lambda v: v.reshape(1, h)

    ts, tr = _make_tables(x, eb_W0[:h], eb_W0[h:2 * h])
    gs = _sc_gather(ts, senders)
    gr = _sc_gather(tr, receivers)
    en, eo = _edge_mlp(gs, gr, edge_attr, eb_W0[2 * h:], r2(eb_b0), eb_W1,
                       r2(eb_b1), eb_W2, r2(eb_b2), eb_W3, r2(eb_b3),
                       r2(eb_g), r2(eb_beta))
    parts, n_pad = _sc_scatter(en, receivers, n)
    xo = _node_mlp(x, parts[:n], parts[n_pad:n_pad + n],
                   nb_W0[:h], nb_W0[h:], r2(nb_b0),
                   nb_W1, r2(nb_b1), nb_W2, r2(nb_b2), nb_W3, r2(nb_b3),
                   r2(nb_g), r2(nb_beta))
    return (xo, eo)
